# Initial kernel scaffold; baseline (speedup 1.0000x reference)
import jax
import jax.numpy as jnp
from jax.experimental import pallas as pl


def kernel(targets, contexts, neg_samples, W_hidden, W_output):
    raise NotImplementedError("write your pallas kernel here")


# speedup vs baseline: 5.3366x; 5.3366x over previous
import functools

import jax
import jax.numpy as jnp
from jax import lax
from jax.experimental import pallas as pl
from jax.experimental.pallas import tpu as pltpu
from jax.experimental.pallas import tpu_sc as plsc

B = 16384
D = 64
K = 20
NW = 32
BPW = B // NW
CH = 64
NCHUNK = BPW // CH
NGI = CH * K // 128


def _sc_scores(tgt, ctx, negf, w_hidden, w_output):
    mesh = plsc.VectorSubcoreMesh(core_axis_name="c", subcore_axis_name="s")

    @functools.partial(
        pl.kernel,
        out_type=(
            jax.ShapeDtypeStruct((B * 16,), jnp.float32),
            jax.ShapeDtypeStruct((B * 16,), jnp.float32),
        ),
        mesh=mesh,
        compiler_params=pltpu.CompilerParams(use_tc_tiling_on_sc=False),
        scratch_types=(
            pltpu.VMEM((CH,), jnp.int32),
            pltpu.VMEM((CH,), jnp.int32),
            pltpu.VMEM((CH * K,), jnp.int32),
            pltpu.VMEM((CH, D), jnp.float32),
            pltpu.VMEM((CH, D), jnp.float32),
            pltpu.VMEM((CH * K, D), jnp.float32),
            pltpu.VMEM((CH * 16,), jnp.float32),
            pltpu.VMEM((CH * 16,), jnp.float32),
            pltpu.SemaphoreType.DMA,
        ),
    )
    def sc_k(tgt_hbm, ctx_hbm, negf_hbm, wh_hbm, wo_hbm, pos_out, neg_out,
             idx_t, idx_c, idx_n, rows_h, rows_o, negbuf,
             sc_p, sc_n, sem):
        wid = lax.axis_index("s") * 2 + lax.axis_index("c")

        def chunk_body(c, carry):
            base = wid * BPW + c * CH
            pltpu.sync_copy(tgt_hbm.at[pl.ds(base, CH)], idx_t)
            pltpu.sync_copy(ctx_hbm.at[pl.ds(base, CH)], idx_c)
            pltpu.sync_copy(negf_hbm.at[pl.ds(base * K, CH * K)], idx_n)
            cps = [
                pltpu.async_copy(wh_hbm.at[idx_t], rows_h, sem),
                pltpu.async_copy(wo_hbm.at[idx_c], rows_o, sem),
            ]
            for i in range(NGI):
                cps.append(pltpu.async_copy(
                    wo_hbm.at[idx_n.at[pl.ds(i * 128, 128)]],
                    negbuf.at[pl.ds(i * 128, 128)], sem))
            for cp in cps:
                cp.wait()

            def b_body(b, carry2):
                h0 = rows_h[b, pl.ds(0, 16)]
                h1 = rows_h[b, pl.ds(16, 16)]
                h2 = rows_h[b, pl.ds(32, 16)]
                h3 = rows_h[b, pl.ds(48, 16)]
                accp = (h0 * rows_o[b, pl.ds(0, 16)]
                        + h1 * rows_o[b, pl.ds(16, 16)]
                        + h2 * rows_o[b, pl.ds(32, 16)]
                        + h3 * rows_o[b, pl.ds(48, 16)])
                accn = jnp.zeros((16,), jnp.float32)
                for j in range(K):
                    r = b * K + j
                    accn = accn + (h0 * negbuf[r, pl.ds(0, 16)]
                                   + h1 * negbuf[r, pl.ds(16, 16)]
                                   + h2 * negbuf[r, pl.ds(32, 16)]
                                   + h3 * negbuf[r, pl.ds(48, 16)])
                sc_p[pl.ds(b * 16, 16)] = accp
                sc_n[pl.ds(b * 16, 16)] = accn
                return carry2

            lax.fori_loop(0, CH, b_body, 0)
            pltpu.sync_copy(sc_p, pos_out.at[pl.ds(base * 16, CH * 16)])
            pltpu.sync_copy(sc_n, neg_out.at[pl.ds(base * 16, CH * 16)])
            return carry

        lax.fori_loop(0, NCHUNK, chunk_body, 0)

    return sc_k(tgt, ctx, negf, w_hidden, w_output)


def _tc_loss(pos_ref, neg_ref, out_ref):
    ri = lax.broadcasted_iota(jnp.int32, (128, 8), 0)
    ci = lax.broadcasted_iota(jnp.int32, (128, 8), 1)
    m = jnp.where(ri // 16 == ci, 1.0, 0.0).astype(jnp.float32)
    sp = jnp.dot(pos_ref[...], m, preferred_element_type=jnp.float32)
    sn = jnp.dot(neg_ref[...], m, preferred_element_type=jnp.float32)
    s = jnp.sum(jax.nn.log_sigmoid(sp)) + jnp.sum(jax.nn.log_sigmoid(-sn))
    out_ref[0, 0] = -s


def kernel(targets, contexts, neg_samples, W_hidden, W_output):
    tgt = targets.astype(jnp.int32)
    ctx = contexts.astype(jnp.int32)
    negf = neg_samples.astype(jnp.int32).reshape(B * K)
    pos, neg = _sc_scores(tgt, ctx, negf, W_hidden, W_output)
    out = pl.pallas_call(
        _tc_loss,
        out_shape=jax.ShapeDtypeStruct((1, 1), jnp.float32),
        out_specs=pl.BlockSpec(memory_space=pltpu.SMEM),
    )(pos.reshape(B // 8, 128), neg.reshape(B // 8, 128))
    return out[0, 0]


# speedup vs baseline: 6.0894x; 1.1411x over previous
import functools

import jax
import jax.numpy as jnp
from jax import lax
from jax.experimental import pallas as pl
from jax.experimental.pallas import tpu as pltpu
from jax.experimental.pallas import tpu_sc as plsc

B = 16384
D = 64
K = 20
NW = 32
BPW = B // NW
CH = 32
NCHUNK = BPW // CH
NGI = CH * K // 128


def _sc_scores(tgt, ctx, negf, w_both):
    mesh = plsc.VectorSubcoreMesh(core_axis_name="c", subcore_axis_name="s")

    @functools.partial(
        pl.kernel,
        out_type=(
            jax.ShapeDtypeStruct((B * 16,), jnp.float32),
            jax.ShapeDtypeStruct((B * 16,), jnp.float32),
        ),
        mesh=mesh,
        scratch_types=(
            pltpu.VMEM((CH,), jnp.int32),
            pltpu.VMEM((CH,), jnp.int32),
            pltpu.VMEM((CH * K,), jnp.int32),
            pltpu.VMEM((CH, 128), jnp.float32),
            pltpu.VMEM((CH, 128), jnp.float32),
            pltpu.VMEM((CH * K, 128), jnp.float32),
            pltpu.VMEM((CH * 16,), jnp.float32),
            pltpu.VMEM((CH * 16,), jnp.float32),
            pltpu.SemaphoreType.DMA,
        ),
    )
    def sc_k(tgt_hbm, ctx_hbm, negf_hbm, wb_hbm, pos_out, neg_out,
             idx_t, idx_c, idx_n, rows_h, rows_o, negbuf, sc_p, sc_n, sem):
        wid = lax.axis_index("s") * 2 + lax.axis_index("c")

        def chunk_body(c, carry):
            base = wid * BPW + c * CH
            pltpu.sync_copy(tgt_hbm.at[pl.ds(base, CH)], idx_t)
            pltpu.sync_copy(ctx_hbm.at[pl.ds(base, CH)], idx_c)
            pltpu.sync_copy(negf_hbm.at[pl.ds(base * K, CH * K)], idx_n)
            cps = [
                pltpu.async_copy(wb_hbm.at[idx_t], rows_h, sem),
                pltpu.async_copy(wb_hbm.at[idx_c], rows_o, sem),
            ]
            for i in range(NGI):
                cps.append(pltpu.async_copy(
                    wb_hbm.at[idx_n.at[pl.ds(i * 128, 128)]],
                    negbuf.at[pl.ds(i * 128, 128)], sem))
            for cp in cps:
                cp.wait()

            def b_body(b, carry2):
                h0 = rows_h[b, pl.ds(0, 16)]
                h1 = rows_h[b, pl.ds(16, 16)]
                h2 = rows_h[b, pl.ds(32, 16)]
                h3 = rows_h[b, pl.ds(48, 16)]
                accp = (h0 * rows_o[b, pl.ds(64, 16)]
                        + h1 * rows_o[b, pl.ds(80, 16)]
                        + h2 * rows_o[b, pl.ds(96, 16)]
                        + h3 * rows_o[b, pl.ds(112, 16)])
                accn = jnp.zeros((16,), jnp.float32)
                for j in range(K):
                    r = b * K + j
                    accn = accn + (h0 * negbuf[r, pl.ds(64, 16)]
                                   + h1 * negbuf[r, pl.ds(80, 16)]
                                   + h2 * negbuf[r, pl.ds(96, 16)]
                                   + h3 * negbuf[r, pl.ds(112, 16)])
                sc_p[pl.ds(b * 16, 16)] = accp
                sc_n[pl.ds(b * 16, 16)] = accn
                return carry2

            lax.fori_loop(0, CH, b_body, 0)
            pltpu.sync_copy(sc_p, pos_out.at[pl.ds(base * 16, CH * 16)])
            pltpu.sync_copy(sc_n, neg_out.at[pl.ds(base * 16, CH * 16)])
            return carry

        lax.fori_loop(0, NCHUNK, chunk_body, 0)

    return sc_k(tgt, ctx, negf, w_both)


def _tc_loss(pos_ref, neg_ref, out_ref):
    ri = lax.broadcasted_iota(jnp.int32, (128, 8), 0)
    ci = lax.broadcasted_iota(jnp.int32, (128, 8), 1)
    m = jnp.where(ri // 16 == ci, 1.0, 0.0).astype(jnp.float32)
    sp = jnp.dot(pos_ref[...], m, preferred_element_type=jnp.float32)
    sn = jnp.dot(neg_ref[...], m, preferred_element_type=jnp.float32)
    s = jnp.sum(jax.nn.log_sigmoid(sp)) + jnp.sum(jax.nn.log_sigmoid(-sn))
    out_ref[0, 0] = -s


def kernel(targets, contexts, neg_samples, W_hidden, W_output):
    tgt = targets.astype(jnp.int32)
    ctx = contexts.astype(jnp.int32)
    negf = neg_samples.astype(jnp.int32).reshape(B * K)
    w_both = jnp.concatenate([W_hidden, W_output], axis=1)
    pos, neg = _sc_scores(tgt, ctx, negf, w_both)
    out = pl.pallas_call(
        _tc_loss,
        out_shape=jax.ShapeDtypeStruct((1, 1), jnp.float32),
        out_specs=pl.BlockSpec(memory_space=pltpu.SMEM),
    )(pos.reshape(B // 8, 128), neg.reshape(B // 8, 128))
    return out[0, 0]
